# baseline (speedup 1.0000x reference)
import functools

import jax
import jax.numpy as jnp
from jax.experimental import pallas as pl
from jax.experimental.pallas import tpu as pltpu

B = 1
S = 2048
HIDDEN = 1024
NUM_EXPERTS = 8
TOP_K = 2
INTER = 2048

NSLOT = S * TOP_K
BLK = 128
NSB = S // BLK
NRB = (NSLOT + 7 * BLK) // BLK
NROWS_R = NRB * BLK
NBLK = NSB + NRB


def _router_body(x_ref, rw_ref, sel_ref, w_ref):
    x = x_ref[...]
    logits = jax.lax.dot_general(
        x, rw_ref[...], (((1,), (1,)), ((), ())),
        preferred_element_type=jnp.float32)
    lane = jax.lax.broadcasted_iota(jnp.int32, (S, NUM_EXPERTS), 1)
    m1 = jnp.max(logits, axis=1, keepdims=True)
    a1 = jnp.min(jnp.where(logits == m1, lane, NUM_EXPERTS), axis=1,
                 keepdims=True)
    l2 = jnp.where(lane == a1, -jnp.inf, logits)
    m2 = jnp.max(l2, axis=1, keepdims=True)
    a2 = jnp.min(jnp.where(l2 == m2, lane, NUM_EXPERTS), axis=1,
                 keepdims=True)
    e2 = jnp.exp(m2 - m1)
    denom = 1.0 + e2
    sel_ref[...] = jnp.concatenate([a1, a2], axis=1)
    w_ref[...] = jnp.concatenate([1.0 / denom, e2 / denom], axis=1)


def _router(x, router_w, interpret=False):
    return pl.pallas_call(
        _router_body,
        out_shape=(jax.ShapeDtypeStruct((S, TOP_K), jnp.int32),
                   jax.ShapeDtypeStruct((S, TOP_K), jnp.float32)),
        interpret=interpret,
    )(x, router_w)


def _dispatch_emul(sel, w):
    sel_flat = sel.reshape(-1)
    w_flat = w.reshape(-1)
    oh = (sel_flat[:, None] == jnp.arange(NUM_EXPERTS)[None, :]).astype(
        jnp.int32)
    counts = jnp.sum(oh, axis=0)
    pc = ((counts + BLK - 1) // BLK) * BLK
    off_end = jnp.cumsum(pc)
    off = off_end - pc
    rank = jnp.cumsum(oh, axis=0) - 1
    rank_j = jnp.sum(rank * oh, axis=1)
    dest = off[sel_flat] + rank_j
    slot_token = jnp.arange(NSLOT, dtype=jnp.int32) // TOP_K
    row_token = jnp.zeros((NROWS_R,), jnp.int32).at[dest].set(slot_token)
    row_weight = jnp.zeros((NROWS_R,), jnp.float32).at[dest].set(w_flat)
    b_iota = jnp.arange(NRB, dtype=jnp.int32) * BLK
    be_routed = jnp.minimum(
        jnp.sum((b_iota[:, None] >= off_end[None, :]).astype(jnp.int32),
                axis=1), NUM_EXPERTS - 1)
    block_expert = jnp.concatenate(
        [jnp.full((NSB,), NUM_EXPERTS, jnp.int32), be_routed])
    return row_token, row_weight, block_expert


def _ffn_body(be_ref, x_ref, xs_ref, gate_ref, up_ref, down_ref, w_ref,
              ys_ref):
    b = pl.program_id(0)
    xin = jnp.where(b < NSB, x_ref[...], xs_ref[...])
    xb = xin.astype(jnp.bfloat16)
    g = jax.lax.dot_general(
        xb, gate_ref[0], (((1,), (1,)), ((), ())),
        preferred_element_type=jnp.float32)
    g = g * jax.lax.logistic(g)
    u = jax.lax.dot_general(
        xb, up_ref[0], (((1,), (1,)), ((), ())),
        preferred_element_type=jnp.float32)
    h = (g * u).astype(jnp.bfloat16)
    y = jax.lax.dot_general(
        h, down_ref[0], (((1,), (1,)), ((), ())),
        preferred_element_type=jnp.float32)
    wcol = jnp.where(b < NSB, 1.0, w_ref[...])
    ys_ref[...] = y * wcol


def _grouped_ffn(block_expert, x, xs, gate_all, up_all, down_all, row_weight,
                 interpret=False):
    grid_spec = pltpu.PrefetchScalarGridSpec(
        num_scalar_prefetch=1,
        grid=(NBLK,),
        in_specs=[
            pl.BlockSpec((BLK, HIDDEN),
                         lambda b, be: (jnp.minimum(b, NSB - 1), 0)),
            pl.BlockSpec((BLK, HIDDEN),
                         lambda b, be: (jnp.maximum(b - NSB, 0), 0)),
            pl.BlockSpec((1, INTER, HIDDEN), lambda b, be: (be[b], 0, 0)),
            pl.BlockSpec((1, INTER, HIDDEN), lambda b, be: (be[b], 0, 0)),
            pl.BlockSpec((1, HIDDEN, INTER), lambda b, be: (be[b], 0, 0)),
            pl.BlockSpec((BLK, 1), lambda b, be: (jnp.maximum(b - NSB, 0), 0)),
        ],
        out_specs=pl.BlockSpec((BLK, HIDDEN), lambda b, be: (b, 0)),
    )
    return pl.pallas_call(
        _ffn_body,
        grid_spec=grid_spec,
        out_shape=jax.ShapeDtypeStruct((NSB * BLK + NROWS_R, HIDDEN),
                                       jnp.float32),
        interpret=interpret,
    )(block_expert, x, xs, gate_all, up_all, down_all,
      row_weight.reshape(NROWS_R, 1))


def kernel(hidden_states, router_w, gate_w, up_w, down_w,
           shared_gate_w, shared_up_w, shared_down_w):
    x = hidden_states.reshape(S, HIDDEN)
    gate_all = jnp.concatenate(
        [gate_w, shared_gate_w[None]], axis=0).astype(jnp.bfloat16)
    up_all = jnp.concatenate(
        [up_w, shared_up_w[None]], axis=0).astype(jnp.bfloat16)
    down_all = jnp.concatenate(
        [down_w, shared_down_w[None]], axis=0).astype(jnp.bfloat16)

    sel, w = _router(x, router_w)
    row_token, row_weight, block_expert = _dispatch_emul(sel, w)
    xs = x[row_token]
    ys = _grouped_ffn(block_expert, x, xs, gate_all, up_all, down_all,
                      row_weight)
    out = ys[:S] + jnp.zeros((S, HIDDEN), jnp.float32).at[row_token].add(
        ys[S:])
    return (out.reshape(B, S, HIDDEN), 0.0)
